# baseline (speedup 1.0000x reference)
import functools

import jax
import jax.numpy as jnp
from jax import lax
from jax.experimental import pallas as pl
from jax.experimental.pallas import tpu as pltpu
from jax.experimental.pallas import tpu_sc as plsc

_SMOOTHING = 0.1
_CONFIDENCE = 1.0 - _SMOOTHING
_PAD = 0
_VOCAB = 32000
_SMOOTH_VAL = _SMOOTHING / (_VOCAB - 2)

_ROW_BLK = 2048
_COL_BLK = 3200

_NC = 2
_NS = 16
_NW = _NC * _NS
_LANES = 16
_SEG = 128


def _fill_kernel(ids_ref, out_ref):
    tgt = ids_ref[:, 0][:, None]
    row_val = jnp.where(tgt == _PAD, 0.0, _SMOOTH_VAL).astype(jnp.float32)
    out_ref[...] = jnp.broadcast_to(row_val, (_ROW_BLK, _COL_BLK))

    @pl.when(pl.program_id(1) == 0)
    def _zero_pad_col():
        out_ref[:, pl.ds(0, 1)] = jnp.zeros((_ROW_BLK, 1), jnp.float32)


def _sc_scatter_body(rows_per_w, ids_hbm, out_ref, ids_v, seg_buf, sem):
    wid = lax.axis_index("s") * _NC + lax.axis_index("c")
    base_row = wid * rows_per_w
    pltpu.sync_copy(ids_hbm.at[pl.ds(base_row, rows_per_w)], ids_v)
    lane = lax.iota(jnp.int32, _LANES)

    @pl.loop(0, rows_per_w)
    def _scatter(i):
        off = pl.multiple_of((i // _LANES) * _LANES, _LANES)
        chunk = ids_v[pl.ds(off, _LANES)]
        col = jnp.max(jnp.where(lane == i % _LANES, chunk, 0))
        pad = col == _PAD
        base_val = jnp.where(pad, 0.0, _SMOOTH_VAL).astype(jnp.float32)
        val = jnp.where(pad, 0.0, _CONFIDENCE).astype(jnp.float32)
        seg = pl.multiple_of((col // _SEG) * _SEG, _SEG)
        within = col % _SEG
        zero_lane = jnp.where(seg == 0, 0, -1)
        for c in range(_SEG // _LANES):
            piece = jnp.where(lane == within - c * _LANES, val, base_val)
            if c == 0:
                piece = jnp.where(lane == zero_lane, 0.0, piece)
            seg_buf[i, pl.ds(c * _LANES, _LANES)] = piece
        pltpu.make_async_copy(
            seg_buf.at[i], out_ref.at[base_row + i, pl.ds(seg, _SEG)], sem
        ).start()

    pltpu.make_async_copy(
        out_ref.at[pl.ds(base_row, rows_per_w), pl.ds(0, _SEG)], seg_buf, sem
    ).wait()


def kernel(trg_token_ids_batch):
    batch = trg_token_ids_batch.shape[0]
    rows_per_w = batch // _NW
    base = pl.pallas_call(
        _fill_kernel,
        grid=(batch // _ROW_BLK, _VOCAB // _COL_BLK),
        in_specs=[pl.BlockSpec((_ROW_BLK, 1), lambda i, j: (i, 0))],
        out_specs=pl.BlockSpec((_ROW_BLK, _COL_BLK), lambda i, j: (i, j)),
        out_shape=jax.ShapeDtypeStruct((batch, _VOCAB), jnp.float32),
    )(trg_token_ids_batch)

    sc_scatter = functools.partial(
        pl.kernel,
        mesh=plsc.VectorSubcoreMesh(
            core_axis_name="c",
            subcore_axis_name="s",
            num_cores=_NC,
            num_subcores=_NS,
        ),
        scratch_types=[
            pltpu.VMEM((rows_per_w,), jnp.int32),
            pltpu.VMEM((rows_per_w, _SEG), jnp.float32),
            pltpu.SemaphoreType.DMA,
        ],
        compiler_params=pltpu.CompilerParams(
            use_tc_tiling_on_sc=True, needs_layout_passes=False
        ),
    )(functools.partial(_sc_scatter_body, rows_per_w))

    out_ref = jax.new_ref(base)
    sc_scatter(trg_token_ids_batch.reshape(batch), out_ref)
    return jax.freeze(out_ref)


# speedup vs baseline: 1.0133x; 1.0133x over previous
import functools

import jax
import jax.numpy as jnp
from jax import lax
from jax.experimental import pallas as pl
from jax.experimental.pallas import tpu as pltpu
from jax.experimental.pallas import tpu_sc as plsc

_SMOOTHING = 0.1
_CONFIDENCE = 1.0 - _SMOOTHING
_PAD = 0
_VOCAB = 32000
_SMOOTH_VAL = _SMOOTHING / (_VOCAB - 2)

_ROW_BLK = 1024
_COL_BLK = 3200

_NC = 2
_NS = 16
_NW = _NC * _NS
_LANES = 16
_SEG = 128


def _fill_kernel(ids_ref, out_ref):
    tgt = ids_ref[:, 0][:, None]
    row_val = jnp.where(tgt == _PAD, 0.0, _SMOOTH_VAL).astype(jnp.float32)
    out_ref[...] = jnp.broadcast_to(row_val, (_ROW_BLK, _COL_BLK))

    @pl.when(pl.program_id(1) == 0)
    def _zero_pad_col():
        out_ref[:, pl.ds(0, 1)] = jnp.zeros((_ROW_BLK, 1), jnp.float32)


def _sc_scatter_body(rows_per_w, ids_hbm, out_ref, ids_v, seg_buf, sem):
    wid = lax.axis_index("s") * _NC + lax.axis_index("c")
    base_row = wid * rows_per_w
    pltpu.sync_copy(ids_hbm.at[pl.ds(base_row, rows_per_w)], ids_v)
    lane = lax.iota(jnp.int32, _LANES)

    @pl.loop(0, rows_per_w // _LANES)
    def _scatter(g):
        off = pl.multiple_of(g * _LANES, _LANES)
        chunk = ids_v[pl.ds(off, _LANES)]
        for l in range(_LANES):
            col = jnp.max(jnp.where(lane == l, chunk, 0))
            i = g * _LANES + l
            pad = col == _PAD
            base_val = jnp.where(pad, 0.0, _SMOOTH_VAL).astype(jnp.float32)
            val = jnp.where(pad, 0.0, _CONFIDENCE).astype(jnp.float32)
            seg = pl.multiple_of((col // _SEG) * _SEG, _SEG)
            within = col % _SEG
            zero_lane = jnp.where(seg == 0, 0, -1)
            for c in range(_SEG // _LANES):
                piece = jnp.where(lane == within - c * _LANES, val, base_val)
                if c == 0:
                    piece = jnp.where(lane == zero_lane, 0.0, piece)
                seg_buf[i, pl.ds(c * _LANES, _LANES)] = piece
            pltpu.make_async_copy(
                seg_buf.at[i], out_ref.at[base_row + i, pl.ds(seg, _SEG)], sem
            ).start()

    pltpu.make_async_copy(
        out_ref.at[pl.ds(base_row, rows_per_w), pl.ds(0, _SEG)], seg_buf, sem
    ).wait()


def kernel(trg_token_ids_batch):
    batch = trg_token_ids_batch.shape[0]
    rows_per_w = batch // _NW
    base = pl.pallas_call(
        _fill_kernel,
        grid=(batch // _ROW_BLK, _VOCAB // _COL_BLK),
        in_specs=[pl.BlockSpec((_ROW_BLK, 1), lambda i, j: (i, 0))],
        out_specs=pl.BlockSpec((_ROW_BLK, _COL_BLK), lambda i, j: (i, j)),
        out_shape=jax.ShapeDtypeStruct((batch, _VOCAB), jnp.float32),
    )(trg_token_ids_batch)

    sc_scatter = functools.partial(
        pl.kernel,
        mesh=plsc.VectorSubcoreMesh(
            core_axis_name="c",
            subcore_axis_name="s",
            num_cores=_NC,
            num_subcores=_NS,
        ),
        scratch_types=[
            pltpu.VMEM((rows_per_w,), jnp.int32),
            pltpu.VMEM((rows_per_w, _SEG), jnp.float32),
            pltpu.SemaphoreType.DMA,
        ],
        compiler_params=pltpu.CompilerParams(
            use_tc_tiling_on_sc=True, needs_layout_passes=False
        ),
    )(functools.partial(_sc_scatter_body, rows_per_w))

    out_ref = jax.new_ref(base)
    sc_scatter(trg_token_ids_batch.reshape(batch), out_ref)
    return jax.freeze(out_ref)
